# baseline (speedup 1.0000x reference)
import functools

import jax
import jax.numpy as jnp
from jax import lax
from jax.experimental import pallas as pl
from jax.experimental.pallas import tpu as pltpu
from jax.experimental.pallas import tpu_sc as plsc

VOCAB = 100000
EMBED = 64
BATCH = 16384

NUM_CORES = 2
NUM_SUBCORES = 16
LANES = 16
NW = NUM_CORES * NUM_SUBCORES
DPW = EMBED // NW
NQ = 4
QSIZE = VOCAB // NQ
NVEC = BATCH // LANES
RSLICE = BATCH // NUM_SUBCORES


def _make_kernel():
  mesh = plsc.VectorSubcoreMesh(core_axis_name="c", subcore_axis_name="s")

  @functools.partial(
      pl.kernel,
      out_type=jax.ShapeDtypeStruct((NUM_CORES, BATCH), jnp.float32),
      mesh=mesh,
      compiler_params=pltpu.CompilerParams(
          needs_layout_passes=False, use_tc_tiling_on_sc=False),
      scratch_types=[
          pltpu.VMEM((1, QSIZE), jnp.float32),
          pltpu.VMEM((1, QSIZE), jnp.float32),
          pltpu.VMEM((BATCH,), jnp.float32),
          pltpu.VMEM((BATCH,), jnp.float32),
          pltpu.VMEM((2048,), jnp.int32),
          pltpu.VMEM((2048,), jnp.int32),
          pltpu.VMEM((BATCH,), jnp.int32),
          pltpu.VMEM_SHARED((NUM_SUBCORES, BATCH), jnp.float32),
          pltpu.SemaphoreType.DMA,
          pltpu.SemaphoreType.DMA,
          pltpu.SemaphoreType.DMA,
          pltpu.SemaphoreType.DMA,
          pltpu.SemaphoreType.DMA,
      ],
  )
  def word2vec_dots(tgt_hbm, ctx_hbm, ttab_hbm, ctab_hbm, out_hbm,
                    col_a, col_b, tv_v, acc_v, tib_a, tib_b, cidx_v,
                    shared, csem_a, csem_b, tsem_a, tsem_b, isem):
    core = lax.axis_index("c")
    sub = lax.axis_index("s")
    lane = lax.iota(jnp.int32, LANES)
    zlane = jnp.zeros((LANES,), jnp.int32)
    zero16 = jnp.zeros((LANES,), jnp.float32)

    cols = (col_a, col_b)
    csems = (csem_a, csem_b)

    tibs = (tib_a, tib_b)
    tsems = (tsem_a, tsem_b)
    TBLK = 2048
    NTB = BATCH // TBLK

    def start_tidx(blk, buf):
      return pltpu.async_copy(
          tgt_hbm.at[pl.ds(blk * TBLK, TBLK)], tibs[buf], tsems[buf])

    ic2 = pltpu.async_copy(ctx_hbm, cidx_v, isem)

    def zero_acc(k, _):
      plsc.store_scatter(acc_v, [k * LANES + lane], zero16)
      return 0

    lax.fori_loop(0, NVEC, zero_acc, 0)
    ic2.wait()

    tasks = []
    for i in range(DPW):
      for tab in (0, 1):
        for q in range(NQ):
          tasks.append((i, tab, q))

    def start_col(t_id):
      i, tab, q = tasks[t_id]
      d = core * (EMBED // NUM_CORES) + sub * DPW + i
      ref = ttab_hbm if tab == 0 else ctab_hbm
      return pltpu.async_copy(
          ref.at[pl.ds(d, 1), pl.ds(q * QSIZE, QSIZE)],
          cols[t_id % 2], csems[t_id % 2])

    pending = start_col(0)
    for t_id, (i, tab, q) in enumerate(tasks):
      pending.wait()
      if t_id + 1 < len(tasks):
        pending = start_col(t_id + 1)
      col = cols[t_id % 2]
      lo = jnp.int32(q * QSIZE)

      if tab == 0:
        pending_tidx = start_tidx(0, 0)
        for blk in range(NTB):
          pending_tidx.wait()
          if blk + 1 < NTB:
            pending_tidx = start_tidx(blk + 1, (blk + 1) % 2)
          tb = tibs[blk % 2]

          def t_body(k, _, col=col, lo=lo, tb=tb, blk=blk):
            iv = tb[pl.ds(k * LANES, LANES)]
            adj = iv - lo
            m = (adj >= 0) & (adj < QSIZE)
            g = plsc.load_gather(col, [zlane, adj], mask=m)
            plsc.store_scatter(tv_v, [jnp.int32(blk * TBLK) + k * LANES + lane],
                               g, mask=m)
            return 0

          lax.fori_loop(0, TBLK // LANES, t_body, 0)
      else:
        def c_body(k, _, col=col, lo=lo):
          iv = cidx_v[pl.ds(k * LANES, LANES)]
          adj = iv - lo
          m = (adj >= 0) & (adj < QSIZE)
          g = plsc.load_gather(col, [zlane, adj], mask=m)
          tvv = tv_v[pl.ds(k * LANES, LANES)]
          plsc.addupdate_scatter(acc_v, [k * LANES + lane], g * tvv, mask=m)
          return 0

        lax.fori_loop(0, NVEC, c_body, 0)

    pltpu.sync_copy(acc_v, shared.at[sub])
    plsc.subcore_barrier()

    base = sub * RSLICE
    for r in range(NUM_SUBCORES):
      pltpu.sync_copy(shared.at[r, pl.ds(base, RSLICE)],
                      tv_v.at[pl.ds(r * RSLICE, RSLICE)])

    def red_body(k, _):
      s = tv_v[pl.ds(k * LANES, LANES)]
      for r in range(1, NUM_SUBCORES):
        s = s + tv_v[pl.ds(r * RSLICE + k * LANES, LANES)]
      plsc.store_scatter(acc_v, [k * LANES + lane], s)
      return 0

    lax.fori_loop(0, RSLICE // LANES, red_body, 0)

    pltpu.sync_copy(acc_v.at[pl.ds(0, RSLICE)],
                    out_hbm.at[core, pl.ds(base, RSLICE)])

  return word2vec_dots


_word2vec_dots = _make_kernel()


@jax.jit
def kernel(target, context, target_table, context_table):
  partials = _word2vec_dots(
      target.astype(jnp.int32), context.astype(jnp.int32),
      target_table.T, context_table.T)
  return (partials[0] + partials[1]).reshape(BATCH, 1)


# speedup vs baseline: 1.7602x; 1.7602x over previous
import functools

import jax
import jax.numpy as jnp
from jax import lax
from jax.experimental import pallas as pl
from jax.experimental.pallas import tpu as pltpu
from jax.experimental.pallas import tpu_sc as plsc

VOCAB = 100000
EMBED = 64
BATCH = 16384

NUM_CORES = 2
NUM_SUBCORES = 16
LANES = 16
NW = NUM_CORES * NUM_SUBCORES
DPW = EMBED // NW
NQ = 4
QSIZE = VOCAB // NQ
NVEC = BATCH // LANES
RSLICE = BATCH // NUM_SUBCORES


def _make_kernel():
  mesh = plsc.VectorSubcoreMesh(core_axis_name="c", subcore_axis_name="s")

  @functools.partial(
      pl.kernel,
      out_type=jax.ShapeDtypeStruct((NUM_CORES, BATCH), jnp.float32),
      mesh=mesh,
      compiler_params=pltpu.CompilerParams(
          needs_layout_passes=False, use_tc_tiling_on_sc=False),
      scratch_types=[
          pltpu.VMEM((1, QSIZE), jnp.float32),
          pltpu.VMEM((1, QSIZE), jnp.float32),
          pltpu.VMEM((BATCH,), jnp.float32),
          pltpu.VMEM((BATCH,), jnp.float32),
          pltpu.VMEM((2048,), jnp.int32),
          pltpu.VMEM((2048,), jnp.int32),
          pltpu.VMEM((BATCH,), jnp.int32),
          pltpu.VMEM_SHARED((NUM_SUBCORES, BATCH), jnp.float32),
          pltpu.SemaphoreType.DMA,
          pltpu.SemaphoreType.DMA,
          pltpu.SemaphoreType.DMA,
          pltpu.SemaphoreType.DMA,
          pltpu.SemaphoreType.DMA,
      ],
  )
  def word2vec_dots(tgt_hbm, ctx_hbm, ttab_hbm, ctab_hbm, out_hbm,
                    col_a, col_b, tv_v, acc_v, tib_a, tib_b, cidx_v,
                    shared, csem_a, csem_b, tsem_a, tsem_b, isem):
    core = lax.axis_index("c")
    sub = lax.axis_index("s")
    lane = lax.iota(jnp.int32, LANES)
    zlane = jnp.zeros((LANES,), jnp.int32)
    zero16 = jnp.zeros((LANES,), jnp.float32)

    cols = (col_a, col_b)
    csems = (csem_a, csem_b)

    tibs = (tib_a, tib_b)
    tsems = (tsem_a, tsem_b)
    TBLK = 2048
    NTB = BATCH // TBLK

    def start_tidx(blk, buf):
      return pltpu.async_copy(
          tgt_hbm.at[pl.ds(blk * TBLK, TBLK)], tibs[buf], tsems[buf])

    ic2 = pltpu.async_copy(ctx_hbm, cidx_v, isem)

    @plsc.parallel_loop(0, NVEC, unroll=8)
    def _(k):
      plsc.store_scatter(acc_v, [k * LANES + lane], zero16)

    ic2.wait()

    tasks = []
    for i in range(DPW):
      for tab in (0, 1):
        for q in range(NQ):
          tasks.append((i, tab, q))

    def start_col(t_id):
      i, tab, q = tasks[t_id]
      d = core * (EMBED // NUM_CORES) + sub * DPW + i
      ref = ttab_hbm if tab == 0 else ctab_hbm
      return pltpu.async_copy(
          ref.at[pl.ds(d, 1), pl.ds(q * QSIZE, QSIZE)],
          cols[t_id % 2], csems[t_id % 2])

    pending = start_col(0)
    for t_id, (i, tab, q) in enumerate(tasks):
      pending.wait()
      if t_id + 1 < len(tasks):
        pending = start_col(t_id + 1)
      col = cols[t_id % 2]
      lo = jnp.int32(q * QSIZE)

      if tab == 0:
        pending_tidx = start_tidx(0, 0)
        for blk in range(NTB):
          pending_tidx.wait()
          if blk + 1 < NTB:
            pending_tidx = start_tidx(blk + 1, (blk + 1) % 2)
          tb = tibs[blk % 2]

          @plsc.parallel_loop(0, TBLK // LANES, unroll=8)
          def _(k, col=col, lo=lo, tb=tb, blk=blk):
            iv = tb[pl.ds(k * LANES, LANES)]
            adj = iv - lo
            m = (adj >= 0) & (adj < QSIZE)
            g = plsc.load_gather(col, [zlane, adj], mask=m)
            plsc.store_scatter(tv_v, [jnp.int32(blk * TBLK) + k * LANES + lane],
                               g, mask=m)
      else:
        @plsc.parallel_loop(0, NVEC, unroll=8)
        def _(k, col=col, lo=lo):
          iv = cidx_v[pl.ds(k * LANES, LANES)]
          adj = iv - lo
          m = (adj >= 0) & (adj < QSIZE)
          g = plsc.load_gather(col, [zlane, adj], mask=m)
          tvv = tv_v[pl.ds(k * LANES, LANES)]
          plsc.addupdate_scatter(acc_v, [k * LANES + lane], g * tvv, mask=m)

    pltpu.sync_copy(acc_v, shared.at[sub])
    plsc.subcore_barrier()

    base = sub * RSLICE
    for r in range(NUM_SUBCORES):
      pltpu.sync_copy(shared.at[r, pl.ds(base, RSLICE)],
                      tv_v.at[pl.ds(r * RSLICE, RSLICE)])

    @plsc.parallel_loop(0, RSLICE // LANES, unroll=4)
    def _(k):
      s = tv_v[pl.ds(k * LANES, LANES)]
      for r in range(1, NUM_SUBCORES):
        s = s + tv_v[pl.ds(r * RSLICE + k * LANES, LANES)]
      plsc.store_scatter(acc_v, [k * LANES + lane], s)

    pltpu.sync_copy(acc_v.at[pl.ds(0, RSLICE)],
                    out_hbm.at[core, pl.ds(base, RSLICE)])

  return word2vec_dots


_word2vec_dots = _make_kernel()


@jax.jit
def kernel(target, context, target_table, context_table):
  partials = _word2vec_dots(
      target.astype(jnp.int32), context.astype(jnp.int32),
      target_table.T, context_table.T)
  return (partials[0] + partials[1]).reshape(BATCH, 1)
